# Initial kernel scaffold; baseline (speedup 1.0000x reference)
import functools

import jax
import jax.numpy as jnp
from jax import lax
from jax.experimental import pallas as pl
from jax.experimental.pallas import tpu as pltpu
from jax.experimental.pallas import tpu_sc as plsc

NUM_EMB = 1000000
DIM = 32
BATCH = 4096
HIST = 200
B_TOTAL = BATCH * HIST

NC = 2
NS = 16
NW = NC * NS
B_W = B_TOTAL // NW

CHUNK = 128
GROUP = 8
G_ROWS = CHUNK * GROUP
N_GROUPS = B_W // G_ROWS

_mesh = plsc.VectorSubcoreMesh(core_axis_name="c", subcore_axis_name="s")


@functools.partial(
    pl.kernel,
    mesh=_mesh,
    out_type=jax.ShapeDtypeStruct((NW, N_GROUPS, G_ROWS, DIM), jnp.float32),
    scratch_types=[
        pltpu.VMEM((GROUP, CHUNK), jnp.int32),
        pltpu.VMEM((G_ROWS, DIM), jnp.float32),
        pltpu.SemaphoreType.DMA,
    ],
)
def _sc_gather(idx_hbm, table_hbm, out_hbm, idx_v, rows_v, sem):
    wid = lax.axis_index("s") * NC + lax.axis_index("c")

    def group_body(g, carry):
        pltpu.sync_copy(idx_hbm.at[wid, g], idx_v)
        copies = []
        for j in range(GROUP):
            copies.append(
                pltpu.async_copy(
                    table_hbm.at[idx_v.at[j]],
                    rows_v.at[pl.ds(j * CHUNK, CHUNK)],
                    sem,
                )
            )
        for c in copies:
            c.wait()
        pltpu.sync_copy(rows_v, out_hbm.at[wid, g])
        return carry

    lax.fori_loop(0, N_GROUPS, group_body, 0)


def kernel(inp, table):
    idx = inp.reshape(NW, N_GROUPS, GROUP, CHUNK)
    out = _sc_gather(idx, table)
    return out.reshape(BATCH, HIST, DIM)


# speedup vs baseline: 1.4581x; 1.4581x over previous
import functools

import jax
import jax.numpy as jnp
from jax import lax
from jax.experimental import pallas as pl
from jax.experimental.pallas import tpu as pltpu
from jax.experimental.pallas import tpu_sc as plsc

NUM_EMB = 1000000
DIM = 32
BATCH = 4096
HIST = 200
B_TOTAL = BATCH * HIST

NC = 2
NS = 16
NW = NC * NS
B_W = B_TOTAL // NW

CHUNK = 128
GROUP = 8
G_ROWS = CHUNK * GROUP
N_GROUPS = B_W // G_ROWS

_mesh = plsc.VectorSubcoreMesh(core_axis_name="c", subcore_axis_name="s")


@functools.partial(
    pl.kernel,
    mesh=_mesh,
    out_type=jax.ShapeDtypeStruct((NW, N_GROUPS, G_ROWS, DIM), jnp.float32),
    scratch_types=[
        pltpu.VMEM((GROUP, CHUNK), jnp.int32),
        pltpu.VMEM((G_ROWS, DIM), jnp.float32),
        pltpu.SemaphoreType.DMA,
    ],
    compiler_params=pltpu.CompilerParams(use_tc_tiling_on_sc=False),
)
def _sc_gather(idx_hbm, table_hbm, out_hbm, idx_v, rows_v, sem):
    wid = lax.axis_index("s") * NC + lax.axis_index("c")

    def group_body(g, carry):
        pltpu.sync_copy(idx_hbm.at[wid, g], idx_v)
        copies = []
        for j in range(GROUP):
            copies.append(
                pltpu.async_copy(
                    table_hbm.at[idx_v.at[j]],
                    rows_v.at[pl.ds(j * CHUNK, CHUNK)],
                    sem,
                )
            )
        for c in copies:
            c.wait()
        pltpu.sync_copy(rows_v, out_hbm.at[wid, g])
        return carry

    lax.fori_loop(0, N_GROUPS, group_body, 0)


def kernel(inp, table):
    idx = inp.reshape(NW, N_GROUPS, GROUP, CHUNK)
    out = _sc_gather(idx, table)
    return out.reshape(BATCH, HIST, DIM)


# speedup vs baseline: 1.5033x; 1.0310x over previous
import functools

import jax
import jax.numpy as jnp
from jax import lax
from jax.experimental import pallas as pl
from jax.experimental.pallas import tpu as pltpu
from jax.experimental.pallas import tpu_sc as plsc

NUM_EMB = 1000000
DIM = 32
BATCH = 4096
HIST = 200
B_TOTAL = BATCH * HIST

NC = 2
NS = 16
NW = NC * NS
B_W = B_TOTAL // NW

CHUNK = 128
GROUP = 5
G_ROWS = CHUNK * GROUP
N_GROUPS = B_W // G_ROWS
NB = 4
N_STREAMS = N_GROUPS * GROUP

_mesh = plsc.VectorSubcoreMesh(core_axis_name="c", subcore_axis_name="s")


@functools.partial(
    pl.kernel,
    mesh=_mesh,
    out_type=jax.ShapeDtypeStruct((NW, N_GROUPS, G_ROWS, DIM), jnp.float32),
    scratch_types=[
        pltpu.VMEM((N_STREAMS, CHUNK), jnp.int32),
        pltpu.VMEM((NB, G_ROWS, DIM), jnp.float32),
        [pltpu.SemaphoreType.DMA] * NB,
        [pltpu.SemaphoreType.DMA] * NB,
    ],
    compiler_params=pltpu.CompilerParams(use_tc_tiling_on_sc=False),
)
def _sc_gather(idx_hbm, table_hbm, out_hbm, idx_v, rows_v, gsems, osems):
    wid = lax.axis_index("s") * NC + lax.axis_index("c")

    def fire_gathers(g, b):
        for j in range(GROUP):
            pltpu.async_copy(
                table_hbm.at[idx_v.at[g * GROUP + j]],
                rows_v.at[b, pl.ds(j * CHUNK, CHUNK)],
                gsems[b],
            )

    def wait_gathers(b):
        pltpu.make_async_copy(
            table_hbm.at[pl.ds(0, G_ROWS)], rows_v.at[b], gsems[b]
        ).wait()

    def wait_write(b):
        pltpu.make_async_copy(
            rows_v.at[b], out_hbm.at[0, 0], osems[b]
        ).wait()

    pltpu.sync_copy(idx_hbm.at[wid], idx_v)

    for b in range(NB - 1):
        fire_gathers(b, b)

    def outer(p, carry):
        for b in range(NB):
            g = p * NB + b
            bf = (b + NB - 1) % NB
            gf = g + NB - 1
            if b == 0:
                @pl.when(p >= 1)
                def _():
                    wait_write(bf)

                fire_gathers(gf, bf)
            else:
                wait_write(bf)

                @pl.when(gf < N_GROUPS)
                def _():
                    fire_gathers(gf, bf)

            wait_gathers(b)
            pltpu.async_copy(rows_v.at[b], out_hbm.at[wid, g], osems[b])
        return carry

    lax.fori_loop(0, N_GROUPS // NB, outer, 0)
    wait_write((N_GROUPS - 1) % NB)


def kernel(inp, table):
    idx = inp.reshape(NW, N_STREAMS, CHUNK)
    out = _sc_gather(idx, table)
    return out.reshape(BATCH, HIST, DIM)
